# baseline (speedup 1.0000x reference)
import functools

import jax
import jax.numpy as jnp
from jax import lax
from jax.experimental import pallas as pl
from jax.experimental.pallas import tpu as pltpu
from jax.experimental.pallas import tpu_sc as plsc

N = 10000
E = 320000
D = 128
DEPTH = 3
BATCH = 10
GPN = N // BATCH

NC = 2
NS = 16
NW = NC * NS
CH = 128
CPW = 80
EPC = NW * CPW * CH
NP = 10240
RPT = NP // NS
ZR = 128
NRING = 4


def _seg_partials(h, src2d, dst2d):
  mesh = plsc.VectorSubcoreMesh(core_axis_name="c", subcore_axis_name="s")

  @functools.partial(
      pl.kernel,
      mesh=mesh,
      out_type=jax.ShapeDtypeStruct((NC * NP, D), jnp.float32),
      scratch_types=[
          pltpu.VMEM((NRING, CH), jnp.int32),
          pltpu.VMEM((NRING, CH), jnp.int32),
          pltpu.VMEM((CH, D), jnp.float32),
          pltpu.VMEM((CH, D), jnp.float32),
          pltpu.VMEM_SHARED((NP, D), jnp.float32),
          pltpu.SemaphoreType.DMA,
          pltpu.SemaphoreType.DMA,
          pltpu.SemaphoreType.DMA,
          pltpu.SemaphoreType.DMA,
          pltpu.SemaphoreType.DMA,
      ],
  )
  def k(h_hbm, src_hbm, dst_hbm, out_hbm,
        sidx, didx, rows0, rows1, acc, sem_i, sem0, sem1, ssem0, ssem1):
    cid = lax.axis_index("c")
    sid = lax.axis_index("s")
    wid = sid * NC + cid
    ebase = wid * CPW * CH

    def ifire(j):
      s = j % NRING
      pltpu.make_async_copy(
          src_hbm.at[pl.ds(ebase + j * CH, CH)], sidx.at[s], sem_i).start()
      pltpu.make_async_copy(
          dst_hbm.at[pl.ds(ebase + j * CH, CH)], didx.at[s], sem_i).start()

    def idrain(j):
      s = j % NRING
      pltpu.make_async_copy(
          src_hbm.at[pl.ds(ebase + j * CH, CH)], sidx.at[s], sem_i).wait()
      pltpu.make_async_copy(
          dst_hbm.at[pl.ds(ebase + j * CH, CH)], didx.at[s], sem_i).wait()

    for j in range(2):
      ifire(j)

    zeros16 = jnp.zeros((16,), jnp.float32)

    def zb(i, carry):
      r = i // (D // 16)
      c = i % (D // 16)
      rows0[r, pl.ds(c * 16, 16)] = zeros16
      return carry

    lax.fori_loop(0, CH * (D // 16), zb, 0)

    def zc(i, carry):
      pltpu.sync_copy(rows0, acc.at[pl.ds(sid * RPT + i * ZR, ZR)])
      return carry

    lax.fori_loop(0, RPT // ZR, zc, 0)
    for j in range(2):
      idrain(j)
    plsc.subcore_barrier()

    def gstart(j, buf, sem):
      pltpu.make_async_copy(h_hbm.at[sidx.at[j % NRING]], buf, sem).start()

    def gwait(j, buf, sem):
      pltpu.make_async_copy(h_hbm.at[sidx.at[j % NRING]], buf, sem).wait()

    def sstart(j, buf, sem):
      pltpu.make_async_copy(buf, acc.at[didx.at[j % NRING]], sem).start(
          add=True)

    def swait(j, buf, sem):
      pltpu.make_async_copy(buf, acc.at[didx.at[j % NRING]], sem).wait()

    gstart(0, rows0, sem0)
    gstart(1, rows1, sem1)
    last = CPW // 2 - 1

    def body(jj, carry):
      j = jj * 2

      @pl.when(jj < last)
      def _():
        ifire(j + 2)
        ifire(j + 3)

      gwait(j, rows0, sem0)
      sstart(j, rows0, ssem0)
      gwait(j + 1, rows1, sem1)
      sstart(j + 1, rows1, ssem1)

      @pl.when(jj < last)
      def _():
        idrain(j + 2)
        idrain(j + 3)

      swait(j, rows0, ssem0)

      @pl.when(jj < last)
      def _():
        gstart(j + 2, rows0, sem0)

      swait(j + 1, rows1, ssem1)

      @pl.when(jj < last)
      def _():
        gstart(j + 3, rows1, sem1)

      return carry

    lax.fori_loop(0, CPW // 2, body, 0)
    plsc.subcore_barrier()

    def wo(i, carry):
      r0 = sid * RPT + i * ZR
      pltpu.sync_copy(acc.at[pl.ds(r0, ZR)], rows0)
      pltpu.sync_copy(rows0, out_hbm.at[pl.ds(cid * NP + r0, ZR)])
      return carry

    lax.fori_loop(0, RPT // ZR, wo, 0)

  return k(h, src2d, dst2d)


def _layer_body(h_ref, parts_ref, w1_ref, b1_ref, g1_ref, bb1_ref,
                w2_ref, b2_ref, g2_ref, bb2_ref, o_ref):
  z = h_ref[...] + parts_ref[0] + parts_ref[1]
  z = jnp.dot(z, w1_ref[...], preferred_element_type=jnp.float32) + b1_ref[...]
  mu = jnp.mean(z, axis=0, keepdims=True)
  var = jnp.mean(jnp.square(z - mu), axis=0, keepdims=True)
  z = (z - mu) * lax.rsqrt(var + 1e-5) * g1_ref[...] + bb1_ref[...]
  z = jnp.maximum(z, 0.0)
  z = jnp.dot(z, w2_ref[...], preferred_element_type=jnp.float32) + b2_ref[...]
  mu = jnp.mean(z, axis=0, keepdims=True)
  var = jnp.mean(jnp.square(z - mu), axis=0, keepdims=True)
  z = (z - mu) * lax.rsqrt(var + 1e-5) * g2_ref[...] + bb2_ref[...]
  o_ref[...] = jnp.maximum(z, 0.0)


def _layer_tc(h, parts, w1, b1, g1, bb1, w2, b2, g2, bb2):
  r = lambda a: a.reshape(1, D)
  wsp = lambda s: pl.BlockSpec(s, lambda i: (0,) * len(s))
  return pl.pallas_call(
      _layer_body,
      grid=(1,),
      in_specs=[
          wsp((N, D)),
          wsp((2, N, D)),
          wsp((D, D)), wsp((1, D)), wsp((1, D)), wsp((1, D)),
          wsp((D, D)), wsp((1, D)), wsp((1, D)), wsp((1, D)),
      ],
      out_specs=wsp((N, D)),
      out_shape=jax.ShapeDtypeStruct((N, D), jnp.float32),
  )(h, parts.reshape(2, NP, D), w1, r(b1), r(g1), r(bb1),
    w2, r(b2), r(g2), r(bb2))


def _head_body(x_ref, x1_ref, x2_ref, x3_ref, jkw_ref, jkb_ref,
               linw_ref, linb_ref, vw_ref, vb_ref, piw_ref, pib_ref,
               pi_ref, v_ref):
  xb = x_ref[0]
  hcat = jnp.concatenate([x1_ref[0], x2_ref[0], x3_ref[0]], axis=1)
  hcat = jnp.dot(hcat, jkw_ref[...],
                 preferred_element_type=jnp.float32) + jkb_ref[...]
  xfull = jnp.concatenate([xb, hcat], axis=1)
  feat = jnp.dot(xfull, linw_ref[...],
                 preferred_element_type=jnp.float32) + linb_ref[...]
  piv = jnp.sum(feat * piw_ref[...], axis=1) + pib_ref[0, 0]
  m = jnp.max(piv)
  lse = jnp.log(jnp.sum(jnp.exp(piv - m))) + m
  pi_ref[0, 0, :] = piv - lse
  fm = jnp.mean(feat, axis=0, keepdims=True)
  v = jnp.dot(fm, vw_ref[...], preferred_element_type=jnp.float32) + vb_ref[...]
  vm = jnp.max(v)
  vlse = jnp.log(jnp.sum(jnp.exp(v - vm))) + vm
  v_ref[0] = v - vlse


def _head_tc(x, x1, x2, x3, jk_w, jk_b, lin_w, lin_b, v_w, v_b, pi_w, pi_b):
  g3 = lambda g: (g, 0, 0)
  w0 = lambda g: (0, 0)
  DD = DEPTH * D
  return pl.pallas_call(
      _head_body,
      grid=(BATCH,),
      in_specs=[
          pl.BlockSpec((1, GPN, D), g3),
          pl.BlockSpec((1, GPN, D), g3),
          pl.BlockSpec((1, GPN, D), g3),
          pl.BlockSpec((1, GPN, D), g3),
          pl.BlockSpec((DD, DD), w0),
          pl.BlockSpec((1, DD), w0),
          pl.BlockSpec((D + DD, 32), w0),
          pl.BlockSpec((1, 32), w0),
          pl.BlockSpec((32, 3), w0),
          pl.BlockSpec((1, 3), w0),
          pl.BlockSpec((1, 32), w0),
          pl.BlockSpec((1, 1), w0),
      ],
      out_specs=[
          pl.BlockSpec((1, 1, GPN), g3),
          pl.BlockSpec((1, 1, 3), g3),
      ],
      out_shape=[
          jax.ShapeDtypeStruct((BATCH, 1, GPN), jnp.float32),
          jax.ShapeDtypeStruct((BATCH, 1, 3), jnp.float32),
      ],
  )(x.reshape(BATCH, GPN, D), x1.reshape(BATCH, GPN, D),
    x2.reshape(BATCH, GPN, D), x3.reshape(BATCH, GPN, D),
    jk_w, jk_b.reshape(1, DD), lin_w, lin_b.reshape(1, 32),
    v_w, v_b.reshape(1, 3), pi_w.reshape(1, 32), pi_b.reshape(1, 1))


def kernel(x, edge_index, batch_size, gin_W1, gin_b1, gin_bn_g, gin_bn_b,
           gin_W2, gin_b2, norm_g, norm_b, jk_W, jk_b, lin_W, lin_b,
           v_W, v_b, pi_W, pi_b):
  pad = EPC - E
  ar = jnp.arange(pad, dtype=jnp.int32)
  src2d = jnp.concatenate([edge_index[0], ar % N])
  dst2d = jnp.concatenate([edge_index[1], N + (ar % (NP - N))])
  h = x
  xs = []
  for i in range(DEPTH):
    parts = _seg_partials(h, src2d, dst2d)
    h = _layer_tc(h, parts, gin_W1[i], gin_b1[i],
                  gin_bn_g[i], gin_bn_b[i], gin_W2[i], gin_b2[i],
                  norm_g[i], norm_b[i])
    xs.append(h)
  pi, v = _head_tc(x, xs[0], xs[1], xs[2], jk_W, jk_b, lin_W, lin_b,
                   v_W, v_b, pi_W, pi_b)
  return (pi.reshape(BATCH, GPN), v.reshape(BATCH, 3))


# speedup vs baseline: 1.2714x; 1.2714x over previous
import functools

import jax
import jax.numpy as jnp
from jax import lax
from jax.experimental import pallas as pl
from jax.experimental.pallas import tpu as pltpu
from jax.experimental.pallas import tpu_sc as plsc

N = 10000
E = 320000
D = 128
DEPTH = 3
BATCH = 10
GPN = N // BATCH

NC = 2
NS = 16
NW = NC * NS
CH = 128
CPW = 80
EPC = NW * CPW * CH
NP = 10240
RPT = NP // NS
ZR = 128
NRING = 4


def _seg_partials(h, src2d, dst2d):
  mesh = plsc.VectorSubcoreMesh(core_axis_name="c", subcore_axis_name="s")

  @functools.partial(
      pl.kernel,
      mesh=mesh,
      out_type=jax.ShapeDtypeStruct((NC * NP, D), jnp.float32),
      scratch_types=[
          pltpu.VMEM((NRING, CH), jnp.int32),
          pltpu.VMEM((NRING, CH), jnp.int32),
          pltpu.VMEM((CH, D), jnp.float32),
          pltpu.VMEM((CH, D), jnp.float32),
          pltpu.VMEM_SHARED((NP, D), jnp.float32),
          pltpu.SemaphoreType.DMA,
          pltpu.SemaphoreType.DMA,
          pltpu.SemaphoreType.DMA,
      ],
  )
  def k(h_hbm, src_hbm, dst_hbm, out_hbm,
        sidx, didx, rows0, rows1, acc, sem_i, sem0, sem1):
    cid = lax.axis_index("c")
    sid = lax.axis_index("s")
    wid = sid * NC + cid
    ebase = wid * CPW * CH

    def ifire(j):
      s = j % NRING
      pltpu.make_async_copy(
          src_hbm.at[pl.ds(ebase + j * CH, CH)], sidx.at[s], sem_i).start()
      pltpu.make_async_copy(
          dst_hbm.at[pl.ds(ebase + j * CH, CH)], didx.at[s], sem_i).start()

    def idrain(j):
      s = j % NRING
      pltpu.make_async_copy(
          src_hbm.at[pl.ds(ebase + j * CH, CH)], sidx.at[s], sem_i).wait()
      pltpu.make_async_copy(
          dst_hbm.at[pl.ds(ebase + j * CH, CH)], didx.at[s], sem_i).wait()

    for j in range(2):
      ifire(j)

    zeros16 = jnp.zeros((16,), jnp.float32)

    def zb(i, carry):
      r = i // (D // 16)
      c = i % (D // 16)
      rows0[r, pl.ds(c * 16, 16)] = zeros16
      return carry

    lax.fori_loop(0, CH * (D // 16), zb, 0)

    def zc(i, carry):
      pltpu.sync_copy(rows0, acc.at[pl.ds(sid * RPT + i * ZR, ZR)])
      return carry

    lax.fori_loop(0, RPT // ZR, zc, 0)
    for j in range(2):
      idrain(j)
    plsc.subcore_barrier()

    def gstart(j, buf, sem):
      pltpu.make_async_copy(h_hbm.at[sidx.at[j % NRING]], buf, sem).start()

    def gwait(j, buf, sem):
      pltpu.make_async_copy(h_hbm.at[sidx.at[j % NRING]], buf, sem).wait()

    gstart(0, rows0, sem0)
    last = CPW // 2 - 1

    def body(jj, carry):
      j = jj * 2

      @pl.when(jj < last)
      def _():
        ifire(j + 2)
        ifire(j + 3)

      gstart(j + 1, rows1, sem1)
      gwait(j, rows0, sem0)
      pltpu.sync_copy(rows0, acc.at[didx.at[j % NRING]], add=True)

      @pl.when(jj < last)
      def _():
        idrain(j + 2)
        idrain(j + 3)
        gstart(j + 2, rows0, sem0)

      gwait(j + 1, rows1, sem1)
      pltpu.sync_copy(rows1, acc.at[didx.at[(j + 1) % NRING]], add=True)
      return carry

    lax.fori_loop(0, CPW // 2, body, 0)
    plsc.subcore_barrier()

    def wo(i, carry):
      r0 = sid * RPT + i * ZR
      pltpu.sync_copy(acc.at[pl.ds(r0, ZR)],
                      out_hbm.at[pl.ds(cid * NP + r0, ZR)])
      return carry

    lax.fori_loop(0, RPT // ZR, wo, 0)

  return k(h, src2d, dst2d)


def _layer_body(h_ref, parts_ref, w1_ref, b1_ref, g1_ref, bb1_ref,
                w2_ref, b2_ref, g2_ref, bb2_ref, o_ref):
  z = h_ref[...] + parts_ref[0] + parts_ref[1]
  z = jnp.dot(z, w1_ref[...], preferred_element_type=jnp.float32) + b1_ref[...]
  mu = jnp.mean(z, axis=0, keepdims=True)
  var = jnp.mean(jnp.square(z - mu), axis=0, keepdims=True)
  z = (z - mu) * lax.rsqrt(var + 1e-5) * g1_ref[...] + bb1_ref[...]
  z = jnp.maximum(z, 0.0)
  z = jnp.dot(z, w2_ref[...], preferred_element_type=jnp.float32) + b2_ref[...]
  mu = jnp.mean(z, axis=0, keepdims=True)
  var = jnp.mean(jnp.square(z - mu), axis=0, keepdims=True)
  z = (z - mu) * lax.rsqrt(var + 1e-5) * g2_ref[...] + bb2_ref[...]
  o_ref[...] = jnp.maximum(z, 0.0)


def _layer_tc(h, parts, w1, b1, g1, bb1, w2, b2, g2, bb2):
  r = lambda a: a.reshape(1, D)
  wsp = lambda s: pl.BlockSpec(s, lambda i: (0,) * len(s))
  return pl.pallas_call(
      _layer_body,
      grid=(1,),
      in_specs=[
          wsp((N, D)),
          wsp((2, N, D)),
          wsp((D, D)), wsp((1, D)), wsp((1, D)), wsp((1, D)),
          wsp((D, D)), wsp((1, D)), wsp((1, D)), wsp((1, D)),
      ],
      out_specs=wsp((N, D)),
      out_shape=jax.ShapeDtypeStruct((N, D), jnp.float32),
  )(h, parts.reshape(2, NP, D), w1, r(b1), r(g1), r(bb1),
    w2, r(b2), r(g2), r(bb2))


def _head_body(x_ref, x1_ref, x2_ref, x3_ref, jkw_ref, jkb_ref,
               linw_ref, linb_ref, vw_ref, vb_ref, piw_ref, pib_ref,
               pi_ref, v_ref):
  xb = x_ref[0]
  hcat = jnp.concatenate([x1_ref[0], x2_ref[0], x3_ref[0]], axis=1)
  hcat = jnp.dot(hcat, jkw_ref[...],
                 preferred_element_type=jnp.float32) + jkb_ref[...]
  xfull = jnp.concatenate([xb, hcat], axis=1)
  feat = jnp.dot(xfull, linw_ref[...],
                 preferred_element_type=jnp.float32) + linb_ref[...]
  piv = jnp.sum(feat * piw_ref[...], axis=1) + pib_ref[0, 0]
  m = jnp.max(piv)
  lse = jnp.log(jnp.sum(jnp.exp(piv - m))) + m
  pi_ref[0, 0, :] = piv - lse
  fm = jnp.mean(feat, axis=0, keepdims=True)
  v = jnp.dot(fm, vw_ref[...], preferred_element_type=jnp.float32) + vb_ref[...]
  vm = jnp.max(v)
  vlse = jnp.log(jnp.sum(jnp.exp(v - vm))) + vm
  v_ref[0] = v - vlse


def _head_tc(x, x1, x2, x3, jk_w, jk_b, lin_w, lin_b, v_w, v_b, pi_w, pi_b):
  g3 = lambda g: (g, 0, 0)
  w0 = lambda g: (0, 0)
  DD = DEPTH * D
  return pl.pallas_call(
      _head_body,
      grid=(BATCH,),
      in_specs=[
          pl.BlockSpec((1, GPN, D), g3),
          pl.BlockSpec((1, GPN, D), g3),
          pl.BlockSpec((1, GPN, D), g3),
          pl.BlockSpec((1, GPN, D), g3),
          pl.BlockSpec((DD, DD), w0),
          pl.BlockSpec((1, DD), w0),
          pl.BlockSpec((D + DD, 32), w0),
          pl.BlockSpec((1, 32), w0),
          pl.BlockSpec((32, 3), w0),
          pl.BlockSpec((1, 3), w0),
          pl.BlockSpec((1, 32), w0),
          pl.BlockSpec((1, 1), w0),
      ],
      out_specs=[
          pl.BlockSpec((1, 1, GPN), g3),
          pl.BlockSpec((1, 1, 3), g3),
      ],
      out_shape=[
          jax.ShapeDtypeStruct((BATCH, 1, GPN), jnp.float32),
          jax.ShapeDtypeStruct((BATCH, 1, 3), jnp.float32),
      ],
  )(x.reshape(BATCH, GPN, D), x1.reshape(BATCH, GPN, D),
    x2.reshape(BATCH, GPN, D), x3.reshape(BATCH, GPN, D),
    jk_w, jk_b.reshape(1, DD), lin_w, lin_b.reshape(1, 32),
    v_w, v_b.reshape(1, 3), pi_w.reshape(1, 32), pi_b.reshape(1, 1))


def kernel(x, edge_index, batch_size, gin_W1, gin_b1, gin_bn_g, gin_bn_b,
           gin_W2, gin_b2, norm_g, norm_b, jk_W, jk_b, lin_W, lin_b,
           v_W, v_b, pi_W, pi_b):
  pad = EPC - E
  ar = jnp.arange(pad, dtype=jnp.int32)
  src2d = jnp.concatenate([edge_index[0], ar % N])
  dst2d = jnp.concatenate([edge_index[1], N + (ar % (NP - N))])
  h = x
  xs = []
  for i in range(DEPTH):
    parts = _seg_partials(h, src2d, dst2d)
    h = _layer_tc(h, parts, gin_W1[i], gin_b1[i],
                  gin_bn_g[i], gin_bn_b[i], gin_W2[i], gin_b2[i],
                  norm_g[i], norm_b[i])
    xs.append(h)
  pi, v = _head_tc(x, xs[0], xs[1], xs[2], jk_W, jk_b, lin_W, lin_b,
                   v_W, v_b, pi_W, pi_b)
  return (pi.reshape(BATCH, GPN), v.reshape(BATCH, 3))


# speedup vs baseline: 1.3175x; 1.0362x over previous
import functools

import jax
import jax.numpy as jnp
from jax import lax
from jax.experimental import pallas as pl
from jax.experimental.pallas import tpu as pltpu
from jax.experimental.pallas import tpu_sc as plsc

N = 10000
E = 320000
D = 128
DEPTH = 3
BATCH = 10
GPN = N // BATCH

NC = 2
NS = 16
NW = NC * NS
CH = 128
CPW = 80
EPC = NW * CPW * CH
NP = 10240
RPT = NP // NS
ZR = 128
NRING = 4


def _seg_partials(h, ei, spad, dpad):
  mesh = plsc.VectorSubcoreMesh(core_axis_name="c", subcore_axis_name="s")

  @functools.partial(
      pl.kernel,
      mesh=mesh,
      out_type=jax.ShapeDtypeStruct((NC * NP, D), jnp.float32),
      scratch_types=[
          pltpu.VMEM((NRING, CH), jnp.int32),
          pltpu.VMEM((NRING, CH), jnp.int32),
          pltpu.VMEM((CH, D), jnp.float32),
          pltpu.VMEM((CH, D), jnp.float32),
          pltpu.VMEM_SHARED((NP, D), jnp.float32),
          pltpu.SemaphoreType.DMA,
          pltpu.SemaphoreType.DMA,
          pltpu.SemaphoreType.DMA,
      ],
  )
  def k(h_hbm, ei_hbm, spad_hbm, dpad_hbm, out_hbm,
        sidx, didx, rows0, rows1, acc, sem_i, sem0, sem1):
    cid = lax.axis_index("c")
    sid = lax.axis_index("s")
    wid = sid * NC + cid
    ebase = wid * CPW * CH

    def ifire(j):
      s = j % NRING
      off = ebase + j * CH

      @pl.when(off < E)
      def _():
        pltpu.make_async_copy(
            ei_hbm.at[0, pl.ds(off, CH)], sidx.at[s], sem_i).start()
        pltpu.make_async_copy(
            ei_hbm.at[1, pl.ds(off, CH)], didx.at[s], sem_i).start()

      @pl.when(off >= E)
      def _():
        pltpu.make_async_copy(
            spad_hbm.at[pl.ds(off - E, CH)], sidx.at[s], sem_i).start()
        pltpu.make_async_copy(
            dpad_hbm.at[pl.ds(off - E, CH)], didx.at[s], sem_i).start()

    def idrain(j):
      s = j % NRING
      pltpu.make_async_copy(
          spad_hbm.at[pl.ds(0, CH)], sidx.at[s], sem_i).wait()
      pltpu.make_async_copy(
          spad_hbm.at[pl.ds(0, CH)], didx.at[s], sem_i).wait()

    for j in range(2):
      ifire(j)

    zeros16 = jnp.zeros((16,), jnp.float32)

    def zb(i, carry):
      r = i // (D // 16)
      c = i % (D // 16)
      rows0[r, pl.ds(c * 16, 16)] = zeros16
      return carry

    lax.fori_loop(0, CH * (D // 16), zb, 0)

    def zc(i, carry):
      pltpu.sync_copy(rows0, acc.at[pl.ds(sid * RPT + i * ZR, ZR)])
      return carry

    lax.fori_loop(0, RPT // ZR, zc, 0)
    for j in range(2):
      idrain(j)
    plsc.subcore_barrier()

    def gstart(j, buf, sem):
      pltpu.make_async_copy(h_hbm.at[sidx.at[j % NRING]], buf, sem).start()

    def gwait(j, buf, sem):
      pltpu.make_async_copy(h_hbm.at[sidx.at[j % NRING]], buf, sem).wait()

    gstart(0, rows0, sem0)
    last = CPW // 2 - 1

    def body(jj, carry):
      j = jj * 2

      @pl.when(jj < last)
      def _():
        ifire(j + 2)
        ifire(j + 3)

      gstart(j + 1, rows1, sem1)
      gwait(j, rows0, sem0)
      pltpu.sync_copy(rows0, acc.at[didx.at[j % NRING]], add=True)

      @pl.when(jj < last)
      def _():
        idrain(j + 2)
        idrain(j + 3)
        gstart(j + 2, rows0, sem0)

      gwait(j + 1, rows1, sem1)
      pltpu.sync_copy(rows1, acc.at[didx.at[(j + 1) % NRING]], add=True)
      return carry

    lax.fori_loop(0, CPW // 2, body, 0)
    plsc.subcore_barrier()

    def wo(i, carry):
      r0 = sid * RPT + i * ZR
      pltpu.sync_copy(acc.at[pl.ds(r0, ZR)],
                      out_hbm.at[pl.ds(cid * NP + r0, ZR)])
      return carry

    lax.fori_loop(0, RPT // ZR, wo, 0)

  return k(h, ei, spad, dpad)


def _layer_body(h_ref, parts_ref, w1_ref, b1_ref, g1_ref, bb1_ref,
                w2_ref, b2_ref, g2_ref, bb2_ref, o_ref):
  z = h_ref[...] + parts_ref[0] + parts_ref[1]
  z = jnp.dot(z, w1_ref[...], preferred_element_type=jnp.float32) + b1_ref[...]
  mu = jnp.mean(z, axis=0, keepdims=True)
  var = jnp.mean(jnp.square(z - mu), axis=0, keepdims=True)
  z = (z - mu) * lax.rsqrt(var + 1e-5) * g1_ref[...] + bb1_ref[...]
  z = jnp.maximum(z, 0.0)
  z = jnp.dot(z, w2_ref[...], preferred_element_type=jnp.float32) + b2_ref[...]
  mu = jnp.mean(z, axis=0, keepdims=True)
  var = jnp.mean(jnp.square(z - mu), axis=0, keepdims=True)
  z = (z - mu) * lax.rsqrt(var + 1e-5) * g2_ref[...] + bb2_ref[...]
  o_ref[...] = jnp.maximum(z, 0.0)


def _layer_tc(h, parts, w1, b1, g1, bb1, w2, b2, g2, bb2):
  r = lambda a: a.reshape(1, D)
  wsp = lambda s: pl.BlockSpec(s, lambda i: (0,) * len(s))
  return pl.pallas_call(
      _layer_body,
      grid=(1,),
      in_specs=[
          wsp((N, D)),
          wsp((2, N, D)),
          wsp((D, D)), wsp((1, D)), wsp((1, D)), wsp((1, D)),
          wsp((D, D)), wsp((1, D)), wsp((1, D)), wsp((1, D)),
      ],
      out_specs=wsp((N, D)),
      out_shape=jax.ShapeDtypeStruct((N, D), jnp.float32),
  )(h, parts.reshape(2, NP, D), w1, r(b1), r(g1), r(bb1),
    w2, r(b2), r(g2), r(bb2))


def _head_body(x_ref, x1_ref, x2_ref, x3_ref, jkw_ref, jkb_ref,
               linw_ref, linb_ref, vw_ref, vb_ref, piw_ref, pib_ref,
               pi_ref, v_ref):
  xb = x_ref[0]
  hcat = jnp.concatenate([x1_ref[0], x2_ref[0], x3_ref[0]], axis=1)
  hcat = jnp.dot(hcat, jkw_ref[...],
                 preferred_element_type=jnp.float32) + jkb_ref[...]
  xfull = jnp.concatenate([xb, hcat], axis=1)
  feat = jnp.dot(xfull, linw_ref[...],
                 preferred_element_type=jnp.float32) + linb_ref[...]
  piv = jnp.sum(feat * piw_ref[...], axis=1) + pib_ref[0, 0]
  m = jnp.max(piv)
  lse = jnp.log(jnp.sum(jnp.exp(piv - m))) + m
  pi_ref[0, 0, :] = piv - lse
  fm = jnp.mean(feat, axis=0, keepdims=True)
  v = jnp.dot(fm, vw_ref[...], preferred_element_type=jnp.float32) + vb_ref[...]
  vm = jnp.max(v)
  vlse = jnp.log(jnp.sum(jnp.exp(v - vm))) + vm
  v_ref[0] = v - vlse


def _head_tc(x, x1, x2, x3, jk_w, jk_b, lin_w, lin_b, v_w, v_b, pi_w, pi_b):
  g3 = lambda g: (g, 0, 0)
  w0 = lambda g: (0, 0)
  DD = DEPTH * D
  return pl.pallas_call(
      _head_body,
      grid=(BATCH,),
      in_specs=[
          pl.BlockSpec((1, GPN, D), g3),
          pl.BlockSpec((1, GPN, D), g3),
          pl.BlockSpec((1, GPN, D), g3),
          pl.BlockSpec((1, GPN, D), g3),
          pl.BlockSpec((DD, DD), w0),
          pl.BlockSpec((1, DD), w0),
          pl.BlockSpec((D + DD, 32), w0),
          pl.BlockSpec((1, 32), w0),
          pl.BlockSpec((32, 3), w0),
          pl.BlockSpec((1, 3), w0),
          pl.BlockSpec((1, 32), w0),
          pl.BlockSpec((1, 1), w0),
      ],
      out_specs=[
          pl.BlockSpec((1, 1, GPN), g3),
          pl.BlockSpec((1, 1, 3), g3),
      ],
      out_shape=[
          jax.ShapeDtypeStruct((BATCH, 1, GPN), jnp.float32),
          jax.ShapeDtypeStruct((BATCH, 1, 3), jnp.float32),
      ],
  )(x.reshape(BATCH, GPN, D), x1.reshape(BATCH, GPN, D),
    x2.reshape(BATCH, GPN, D), x3.reshape(BATCH, GPN, D),
    jk_w, jk_b.reshape(1, DD), lin_w, lin_b.reshape(1, 32),
    v_w, v_b.reshape(1, 3), pi_w.reshape(1, 32), pi_b.reshape(1, 1))


def kernel(x, edge_index, batch_size, gin_W1, gin_b1, gin_bn_g, gin_bn_b,
           gin_W2, gin_b2, norm_g, norm_b, jk_W, jk_b, lin_W, lin_b,
           v_W, v_b, pi_W, pi_b):
  pad = EPC - E
  ar = jnp.arange(pad, dtype=jnp.int32)
  spad = ar % N
  dpad = N + (ar % (NP - N))
  h = x
  xs = []
  for i in range(DEPTH):
    parts = _seg_partials(h, edge_index, spad, dpad)
    h = _layer_tc(h, parts, gin_W1[i], gin_b1[i],
                  gin_bn_g[i], gin_bn_b[i], gin_W2[i], gin_b2[i],
                  norm_g[i], norm_b[i])
    xs.append(h)
  pi, v = _head_tc(x, xs[0], xs[1], xs[2], jk_W, jk_b, lin_W, lin_b,
                   v_W, v_b, pi_W, pi_b)
  return (pi.reshape(BATCH, GPN), v.reshape(BATCH, 3))


# speedup vs baseline: 1.3423x; 1.0188x over previous
import functools

import jax
import jax.numpy as jnp
from jax import lax
from jax.experimental import pallas as pl
from jax.experimental.pallas import tpu as pltpu
from jax.experimental.pallas import tpu_sc as plsc

N = 10000
E = 320000
D = 128
DEPTH = 3
BATCH = 10
NV = 3
GPN = N // BATCH

NC = 2
NS = 16
NW = NC * NS
CH = 128
CPW = 80
EPC = NW * CPW * CH
NP = 10240
RPT = NP // NS
ZR = 128
NRING = 4


def _seg_partials(h, ei, spad, dpad):
  mesh = plsc.VectorSubcoreMesh(core_axis_name="c", subcore_axis_name="s")

  @functools.partial(
      pl.kernel,
      mesh=mesh,
      out_type=jax.ShapeDtypeStruct((NC * NP, D), jnp.float32),
      scratch_types=[
          pltpu.VMEM((NRING, CH), jnp.int32),
          pltpu.VMEM((NRING, CH), jnp.int32),
          pltpu.VMEM((CH, D), jnp.float32),
          pltpu.VMEM((CH, D), jnp.float32),
          pltpu.VMEM_SHARED((NP, D), jnp.float32),
          pltpu.SemaphoreType.DMA,
          pltpu.SemaphoreType.DMA,
          pltpu.SemaphoreType.DMA,
      ],
  )
  def k(h_hbm, ei_hbm, spad_hbm, dpad_hbm, out_hbm,
        sidx, didx, rows0, rows1, acc, sem_i, sem0, sem1):
    cid = lax.axis_index("c")
    sid = lax.axis_index("s")
    wid = sid * NC + cid
    ebase = wid * CPW * CH

    def ifire(j):
      s = j % NRING
      off = ebase + j * CH

      @pl.when(off < E)
      def _():
        pltpu.make_async_copy(
            ei_hbm.at[0, pl.ds(off, CH)], sidx.at[s], sem_i).start()
        pltpu.make_async_copy(
            ei_hbm.at[1, pl.ds(off, CH)], didx.at[s], sem_i).start()

      @pl.when(off >= E)
      def _():
        pltpu.make_async_copy(
            spad_hbm.at[pl.ds(off - E, CH)], sidx.at[s], sem_i).start()
        pltpu.make_async_copy(
            dpad_hbm.at[pl.ds(off - E, CH)], didx.at[s], sem_i).start()

    def idrain(j):
      s = j % NRING
      pltpu.make_async_copy(
          spad_hbm.at[pl.ds(0, CH)], sidx.at[s], sem_i).wait()
      pltpu.make_async_copy(
          spad_hbm.at[pl.ds(0, CH)], didx.at[s], sem_i).wait()

    for j in range(2):
      ifire(j)

    zeros16 = jnp.zeros((16,), jnp.float32)

    def zb(i, carry):
      r = i // (D // 16)
      c = i % (D // 16)
      rows0[r, pl.ds(c * 16, 16)] = zeros16
      return carry

    lax.fori_loop(0, CH * (D // 16), zb, 0)

    def zc(i, carry):
      pltpu.sync_copy(rows0, acc.at[pl.ds(sid * RPT + i * ZR, ZR)])
      return carry

    lax.fori_loop(0, RPT // ZR, zc, 0)
    for j in range(2):
      idrain(j)
    plsc.subcore_barrier()

    def gstart(j, buf, sem):
      pltpu.make_async_copy(h_hbm.at[sidx.at[j % NRING]], buf, sem).start()

    def gwait(j, buf, sem):
      pltpu.make_async_copy(h_hbm.at[sidx.at[j % NRING]], buf, sem).wait()

    gstart(0, rows0, sem0)
    last = CPW // 2 - 1

    def body(jj, carry):
      j = jj * 2

      @pl.when(jj < last)
      def _():
        ifire(j + 2)
        ifire(j + 3)

      gstart(j + 1, rows1, sem1)
      gwait(j, rows0, sem0)
      pltpu.sync_copy(rows0, acc.at[didx.at[j % NRING]], add=True)

      @pl.when(jj < last)
      def _():
        idrain(j + 2)
        idrain(j + 3)
        gstart(j + 2, rows0, sem0)

      gwait(j + 1, rows1, sem1)
      pltpu.sync_copy(rows1, acc.at[didx.at[(j + 1) % NRING]], add=True)
      return carry

    lax.fori_loop(0, CPW // 2, body, 0)
    plsc.subcore_barrier()

    def wo(i, carry):
      r0 = sid * RPT + i * ZR
      pltpu.sync_copy(acc.at[pl.ds(r0, ZR)],
                      out_hbm.at[pl.ds(cid * NP + r0, ZR)])
      return carry

    lax.fori_loop(0, RPT // ZR, wo, 0)

  return k(h, ei, spad, dpad)


def _layer_body(h_ref, parts_ref, w1_ref, b1_ref, g1_ref, bb1_ref,
                w2_ref, b2_ref, g2_ref, bb2_ref, qa_ref, qb_ref,
                o_ref, pp_ref, mi_ref, mo_ref):
  hin = h_ref[...]
  z = hin + parts_ref[0] + parts_ref[1]
  z = jnp.dot(z, w1_ref[...], preferred_element_type=jnp.float32) + b1_ref[...]
  mu = jnp.mean(z, axis=0, keepdims=True)
  var = jnp.mean(jnp.square(z - mu), axis=0, keepdims=True)
  z = (z - mu) * lax.rsqrt(var + 1e-5) * g1_ref[...] + bb1_ref[...]
  z = jnp.maximum(z, 0.0)
  z = jnp.dot(z, w2_ref[...], preferred_element_type=jnp.float32) + b2_ref[...]
  mu = jnp.mean(z, axis=0, keepdims=True)
  var = jnp.mean(jnp.square(z - mu), axis=0, keepdims=True)
  z = (z - mu) * lax.rsqrt(var + 1e-5) * g2_ref[...] + bb2_ref[...]
  hout = jnp.maximum(z, 0.0)
  o_ref[...] = hout
  pp_ref[...] = (
      jnp.dot(hin, qa_ref[...], preferred_element_type=jnp.float32)
      + jnp.dot(hout, qb_ref[...], preferred_element_type=jnp.float32))
  mi_ref[...] = jnp.mean(hin.reshape(BATCH, GPN, D), axis=1)
  mo_ref[...] = jnp.mean(hout.reshape(BATCH, GPN, D), axis=1)


def _layer_tc(h, parts, w1, b1, g1, bb1, w2, b2, g2, bb2, qa, qb):
  r = lambda a: a.reshape(1, D)
  wsp = lambda s: pl.BlockSpec(s, lambda i: (0,) * len(s))
  return pl.pallas_call(
      _layer_body,
      grid=(1,),
      in_specs=[
          wsp((N, D)),
          wsp((2, N, D)),
          wsp((D, D)), wsp((1, D)), wsp((1, D)), wsp((1, D)),
          wsp((D, D)), wsp((1, D)), wsp((1, D)), wsp((1, D)),
          wsp((D, 1)), wsp((D, 1)),
      ],
      out_specs=[wsp((N, D)), wsp((N, 1)), wsp((BATCH, D)), wsp((BATCH, D))],
      out_shape=[
          jax.ShapeDtypeStruct((N, D), jnp.float32),
          jax.ShapeDtypeStruct((N, 1), jnp.float32),
          jax.ShapeDtypeStruct((BATCH, D), jnp.float32),
          jax.ShapeDtypeStruct((BATCH, D), jnp.float32),
      ],
  )(h, parts.reshape(2, NP, D), w1, r(b1), r(g1), r(bb1),
    w2, r(b2), r(g2), r(bb2), qa, qb)


def _head_body(pp1_ref, pp2_ref, pp3_ref, mx_ref, m1_ref, m2_ref, m3_ref,
               v0_ref, v1_ref, v2_ref, v3_ref, cpi_ref, cv_ref,
               pi_ref, v_ref):
  s = pp1_ref[...] + pp2_ref[...] + pp3_ref[...] + cpi_ref[0, 0]
  m = jnp.max(s, axis=1, keepdims=True)
  lse = jnp.log(jnp.sum(jnp.exp(s - m), axis=1, keepdims=True)) + m
  pi_ref[...] = s - lse
  dot = lambda a, b: jnp.dot(a, b, preferred_element_type=jnp.float32)
  v = (dot(mx_ref[...], v0_ref[...]) + dot(m1_ref[...], v1_ref[...])
       + dot(m2_ref[...], v2_ref[...]) + dot(m3_ref[...], v3_ref[...])
       + cv_ref[...])
  vm = jnp.max(v, axis=1, keepdims=True)
  vlse = jnp.log(jnp.sum(jnp.exp(v - vm), axis=1, keepdims=True)) + vm
  v_ref[...] = v - vlse


def _head_tc(pp1, pp2, pp3, mx, m1, m2, m3, v0, v1, v2, v3, c_pi, c_v):
  wsp = lambda s: pl.BlockSpec(s, lambda i: (0,) * len(s))
  return pl.pallas_call(
      _head_body,
      grid=(1,),
      in_specs=[
          wsp((BATCH, GPN)), wsp((BATCH, GPN)), wsp((BATCH, GPN)),
          wsp((BATCH, D)), wsp((BATCH, D)), wsp((BATCH, D)), wsp((BATCH, D)),
          wsp((D, NV)), wsp((D, NV)), wsp((D, NV)), wsp((D, NV)),
          wsp((1, 1)), wsp((1, NV)),
      ],
      out_specs=[wsp((BATCH, GPN)), wsp((BATCH, NV))],
      out_shape=[
          jax.ShapeDtypeStruct((BATCH, GPN), jnp.float32),
          jax.ShapeDtypeStruct((BATCH, NV), jnp.float32),
      ],
  )(pp1.reshape(BATCH, GPN), pp2.reshape(BATCH, GPN), pp3.reshape(BATCH, GPN),
    mx, m1, m2, m3, v0, v1, v2, v3, c_pi.reshape(1, 1), c_v.reshape(1, NV))


def kernel(x, edge_index, batch_size, gin_W1, gin_b1, gin_bn_g, gin_bn_b,
           gin_W2, gin_b2, norm_g, norm_b, jk_W, jk_b, lin_W, lin_b,
           v_W, v_b, pi_W, pi_b):
  pad = EPC - E
  ar = jnp.arange(pad, dtype=jnp.int32)
  spad = ar % N
  dpad = N + (ar % (NP - N))

  p_h = jk_W @ (lin_W[D:] @ pi_W)
  q0 = lin_W[:D] @ pi_W
  qs = [p_h[i * D:(i + 1) * D] for i in range(DEPTH)]
  c_pi = jk_b @ (lin_W[D:] @ pi_W) + lin_b @ pi_W + pi_b
  vh = jk_W @ (lin_W[D:] @ v_W)
  v0 = lin_W[:D] @ v_W
  vs = [vh[i * D:(i + 1) * D] for i in range(DEPTH)]
  c_v = jk_b @ (lin_W[D:] @ v_W) + lin_b @ v_W + v_b

  qz = jnp.zeros((D, 1), jnp.float32)
  h = x
  pps, mos = [], []
  mx = None
  for i in range(DEPTH):
    parts = _seg_partials(h, edge_index, spad, dpad)
    qa = q0 if i == 0 else qz
    h, pp, mi, mo = _layer_tc(h, parts, gin_W1[i], gin_b1[i],
                              gin_bn_g[i], gin_bn_b[i], gin_W2[i], gin_b2[i],
                              norm_g[i], norm_b[i], qa, qs[i])
    pps.append(pp)
    mos.append(mo)
    if i == 0:
      mx = mi
  pi, v = _head_tc(pps[0], pps[1], pps[2], mx, mos[0], mos[1], mos[2],
                   v0, vs[0], vs[1], vs[2], c_pi, c_v)
  return (pi, v)


# speedup vs baseline: 1.3532x; 1.0081x over previous
import functools

import jax
import jax.numpy as jnp
from jax import lax
from jax.experimental import pallas as pl
from jax.experimental.pallas import tpu as pltpu
from jax.experimental.pallas import tpu_sc as plsc

N = 10000
E = 320000
D = 128
DEPTH = 3
BATCH = 10
NV = 3
GPN = N // BATCH

NC = 2
NS = 16
NW = NC * NS
CH = 128
CPW = 80
EPC = NW * CPW * CH
NP = 10240
RPT = NP // NS
ZR = 128
NRING = 4


def _seg_partials(h, ei, spad, dpad):
  mesh = plsc.VectorSubcoreMesh(core_axis_name="c", subcore_axis_name="s")

  @functools.partial(
      pl.kernel,
      mesh=mesh,
      out_type=jax.ShapeDtypeStruct((NC * NP, D), jnp.float32),
      scratch_types=[
          pltpu.VMEM((NRING, CH), jnp.int32),
          pltpu.VMEM((NRING, CH), jnp.int32),
          pltpu.VMEM((CH, D), jnp.float32),
          pltpu.VMEM((CH, D), jnp.float32),
          pltpu.VMEM_SHARED((NP, D), jnp.float32),
          pltpu.SemaphoreType.DMA,
          pltpu.SemaphoreType.DMA,
          pltpu.SemaphoreType.DMA,
      ],
  )
  def k(h_hbm, ei_hbm, spad_hbm, dpad_hbm, out_hbm,
        sidx, didx, rows0, rows1, acc, sem_i, sem0, sem1):
    cid = lax.axis_index("c")
    sid = lax.axis_index("s")
    wid = sid * NC + cid
    ebase = wid * CPW * CH

    def ifire(j):
      s = j % NRING
      off = ebase + j * CH

      @pl.when(off < E)
      def _():
        pltpu.make_async_copy(
            ei_hbm.at[0, pl.ds(off, CH)], sidx.at[s], sem_i).start()
        pltpu.make_async_copy(
            ei_hbm.at[1, pl.ds(off, CH)], didx.at[s], sem_i).start()

      @pl.when(off >= E)
      def _():
        pltpu.make_async_copy(
            spad_hbm.at[pl.ds(off - E, CH)], sidx.at[s], sem_i).start()
        pltpu.make_async_copy(
            dpad_hbm.at[pl.ds(off - E, CH)], didx.at[s], sem_i).start()

    def idrain(j):
      s = j % NRING
      pltpu.make_async_copy(
          spad_hbm.at[pl.ds(0, CH)], sidx.at[s], sem_i).wait()
      pltpu.make_async_copy(
          spad_hbm.at[pl.ds(0, CH)], didx.at[s], sem_i).wait()

    for j in range(2):
      ifire(j)

    @pl.when(cid == 0)
    def _():

      def hc(i, carry):
        r0 = sid * RPT + i * ZR
        pltpu.sync_copy(h_hbm.at[pl.ds(r0, ZR)], acc.at[pl.ds(r0, ZR)])
        return carry

      @pl.when(sid < NS - 1)
      def _():
        lax.fori_loop(0, RPT // ZR, hc, 0)

      @pl.when(sid == NS - 1)
      def _():
        lax.fori_loop(0, (N - (NS - 1) * RPT) // ZR, hc, 0)
        nrem = N - (NS - 1) * RPT - ((N - (NS - 1) * RPT) // ZR) * ZR
        if nrem:
          base = (NS - 1) * RPT + ((N - (NS - 1) * RPT) // ZR) * ZR
          pltpu.sync_copy(h_hbm.at[pl.ds(base, nrem)],
                          acc.at[pl.ds(base, nrem)])

    @pl.when(cid == 1)
    def _():
      zeros16 = jnp.zeros((16,), jnp.float32)

      def zb(i, carry):
        r = i // (D // 16)
        c = i % (D // 16)
        rows0[r, pl.ds(c * 16, 16)] = zeros16
        return carry

      lax.fori_loop(0, CH * (D // 16), zb, 0)

      def zc(i, carry):
        pltpu.sync_copy(rows0, acc.at[pl.ds(sid * RPT + i * ZR, ZR)])
        return carry

      lax.fori_loop(0, RPT // ZR, zc, 0)

    for j in range(2):
      idrain(j)
    plsc.subcore_barrier()

    def gstart(j, buf, sem):
      pltpu.make_async_copy(h_hbm.at[sidx.at[j % NRING]], buf, sem).start()

    def gwait(j, buf, sem):
      pltpu.make_async_copy(h_hbm.at[sidx.at[j % NRING]], buf, sem).wait()

    gstart(0, rows0, sem0)
    last = CPW // 2 - 1

    def body(jj, carry):
      j = jj * 2

      @pl.when(jj < last)
      def _():
        ifire(j + 2)
        ifire(j + 3)

      gstart(j + 1, rows1, sem1)
      gwait(j, rows0, sem0)
      pltpu.sync_copy(rows0, acc.at[didx.at[j % NRING]], add=True)

      @pl.when(jj < last)
      def _():
        idrain(j + 2)
        idrain(j + 3)
        gstart(j + 2, rows0, sem0)

      gwait(j + 1, rows1, sem1)
      pltpu.sync_copy(rows1, acc.at[didx.at[(j + 1) % NRING]], add=True)
      return carry

    lax.fori_loop(0, CPW // 2, body, 0)
    plsc.subcore_barrier()

    def wo(i, carry):
      r0 = sid * RPT + i * ZR
      pltpu.sync_copy(acc.at[pl.ds(r0, ZR)],
                      out_hbm.at[pl.ds(cid * NP + r0, ZR)])
      return carry

    lax.fori_loop(0, RPT // ZR, wo, 0)

  return k(h, ei, spad, dpad)


def _mlp(z, w1_ref, b1_ref, g1_ref, bb1_ref, w2_ref, b2_ref, g2_ref, bb2_ref):
  z = jnp.dot(z, w1_ref[...], preferred_element_type=jnp.float32) + b1_ref[...]
  mu = jnp.mean(z, axis=0, keepdims=True)
  var = jnp.mean(jnp.square(z - mu), axis=0, keepdims=True)
  z = (z - mu) * lax.rsqrt(var + 1e-5) * g1_ref[...] + bb1_ref[...]
  z = jnp.maximum(z, 0.0)
  z = jnp.dot(z, w2_ref[...], preferred_element_type=jnp.float32) + b2_ref[...]
  mu = jnp.mean(z, axis=0, keepdims=True)
  var = jnp.mean(jnp.square(z - mu), axis=0, keepdims=True)
  z = (z - mu) * lax.rsqrt(var + 1e-5) * g2_ref[...] + bb2_ref[...]
  return jnp.maximum(z, 0.0)


def _layer1_body(h_ref, parts_ref, w1_ref, b1_ref, g1_ref, bb1_ref,
                 w2_ref, b2_ref, g2_ref, bb2_ref, qa_ref, qb_ref,
                 o_ref, pp_ref, mi_ref, mo_ref):
  hin = h_ref[...]
  hout = _mlp(parts_ref[0] + parts_ref[1], w1_ref, b1_ref, g1_ref, bb1_ref,
              w2_ref, b2_ref, g2_ref, bb2_ref)
  o_ref[...] = hout
  pp_ref[...] = (
      jnp.dot(hin, qa_ref[...], preferred_element_type=jnp.float32)
      + jnp.dot(hout, qb_ref[...], preferred_element_type=jnp.float32))
  mi_ref[...] = jnp.mean(hin.reshape(BATCH, GPN, D), axis=1)
  mo_ref[...] = jnp.mean(hout.reshape(BATCH, GPN, D), axis=1)


def _layer23_body(parts_ref, w1_ref, b1_ref, g1_ref, bb1_ref,
                  w2_ref, b2_ref, g2_ref, bb2_ref, qb_ref,
                  o_ref, pp_ref, mo_ref):
  hout = _mlp(parts_ref[0] + parts_ref[1], w1_ref, b1_ref, g1_ref, bb1_ref,
              w2_ref, b2_ref, g2_ref, bb2_ref)
  o_ref[...] = hout
  pp_ref[...] = jnp.dot(hout, qb_ref[...], preferred_element_type=jnp.float32)
  mo_ref[...] = jnp.mean(hout.reshape(BATCH, GPN, D), axis=1)


def _wsp(s):
  return pl.BlockSpec(s, lambda i: (0,) * len(s))


_WSPECS = [_wsp((D, D)), _wsp((1, D)), _wsp((1, D)), _wsp((1, D)),
           _wsp((D, D)), _wsp((1, D)), _wsp((1, D)), _wsp((1, D))]


def _layer1_tc(h, parts, w1, b1, g1, bb1, w2, b2, g2, bb2, qa, qb):
  r = lambda a: a.reshape(1, D)
  return pl.pallas_call(
      _layer1_body,
      grid=(1,),
      in_specs=[_wsp((N, D)), _wsp((2, N, D))] + _WSPECS
      + [_wsp((D, 1)), _wsp((D, 1))],
      out_specs=[_wsp((N, D)), _wsp((N, 1)), _wsp((BATCH, D)),
                 _wsp((BATCH, D))],
      out_shape=[
          jax.ShapeDtypeStruct((N, D), jnp.float32),
          jax.ShapeDtypeStruct((N, 1), jnp.float32),
          jax.ShapeDtypeStruct((BATCH, D), jnp.float32),
          jax.ShapeDtypeStruct((BATCH, D), jnp.float32),
      ],
  )(h, parts.reshape(2, NP, D), w1, r(b1), r(g1), r(bb1),
    w2, r(b2), r(g2), r(bb2), qa, qb)


def _layer23_tc(parts, w1, b1, g1, bb1, w2, b2, g2, bb2, qb):
  r = lambda a: a.reshape(1, D)
  return pl.pallas_call(
      _layer23_body,
      grid=(1,),
      in_specs=[_wsp((2, N, D))] + _WSPECS + [_wsp((D, 1))],
      out_specs=[_wsp((N, D)), _wsp((N, 1)), _wsp((BATCH, D))],
      out_shape=[
          jax.ShapeDtypeStruct((N, D), jnp.float32),
          jax.ShapeDtypeStruct((N, 1), jnp.float32),
          jax.ShapeDtypeStruct((BATCH, D), jnp.float32),
      ],
  )(parts.reshape(2, NP, D), w1, r(b1), r(g1), r(bb1),
    w2, r(b2), r(g2), r(bb2), qb)


def _head_body(pp1_ref, pp2_ref, pp3_ref, mx_ref, m1_ref, m2_ref, m3_ref,
               v0_ref, v1_ref, v2_ref, v3_ref, cpi_ref, cv_ref,
               pi_ref, v_ref):
  s = pp1_ref[...] + pp2_ref[...] + pp3_ref[...] + cpi_ref[0, 0]
  m = jnp.max(s, axis=1, keepdims=True)
  lse = jnp.log(jnp.sum(jnp.exp(s - m), axis=1, keepdims=True)) + m
  pi_ref[...] = s - lse
  dot = lambda a, b: jnp.dot(a, b, preferred_element_type=jnp.float32)
  v = (dot(mx_ref[...], v0_ref[...]) + dot(m1_ref[...], v1_ref[...])
       + dot(m2_ref[...], v2_ref[...]) + dot(m3_ref[...], v3_ref[...])
       + cv_ref[...])
  vm = jnp.max(v, axis=1, keepdims=True)
  vlse = jnp.log(jnp.sum(jnp.exp(v - vm), axis=1, keepdims=True)) + vm
  v_ref[...] = v - vlse


def _head_tc(pp1, pp2, pp3, mx, m1, m2, m3, v0, v1, v2, v3, c_pi, c_v):
  wsp = lambda s: pl.BlockSpec(s, lambda i: (0,) * len(s))
  return pl.pallas_call(
      _head_body,
      grid=(1,),
      in_specs=[
          wsp((BATCH, GPN)), wsp((BATCH, GPN)), wsp((BATCH, GPN)),
          wsp((BATCH, D)), wsp((BATCH, D)), wsp((BATCH, D)), wsp((BATCH, D)),
          wsp((D, NV)), wsp((D, NV)), wsp((D, NV)), wsp((D, NV)),
          wsp((1, 1)), wsp((1, NV)),
      ],
      out_specs=[wsp((BATCH, GPN)), wsp((BATCH, NV))],
      out_shape=[
          jax.ShapeDtypeStruct((BATCH, GPN), jnp.float32),
          jax.ShapeDtypeStruct((BATCH, NV), jnp.float32),
      ],
  )(pp1.reshape(BATCH, GPN), pp2.reshape(BATCH, GPN), pp3.reshape(BATCH, GPN),
    mx, m1, m2, m3, v0, v1, v2, v3, c_pi.reshape(1, 1), c_v.reshape(1, NV))


def kernel(x, edge_index, batch_size, gin_W1, gin_b1, gin_bn_g, gin_bn_b,
           gin_W2, gin_b2, norm_g, norm_b, jk_W, jk_b, lin_W, lin_b,
           v_W, v_b, pi_W, pi_b):
  pad = EPC - E
  ar = jnp.arange(pad, dtype=jnp.int32)
  spad = ar % N
  dpad = N + (ar % (NP - N))

  p_h = jk_W @ (lin_W[D:] @ pi_W)
  q0 = lin_W[:D] @ pi_W
  qs = [p_h[i * D:(i + 1) * D] for i in range(DEPTH)]
  c_pi = jk_b @ (lin_W[D:] @ pi_W) + lin_b @ pi_W + pi_b
  vh = jk_W @ (lin_W[D:] @ v_W)
  v0 = lin_W[:D] @ v_W
  vs = [vh[i * D:(i + 1) * D] for i in range(DEPTH)]
  c_v = jk_b @ (lin_W[D:] @ v_W) + lin_b @ v_W + v_b

  h = x
  pps, mos = [], []
  mx = None
  for i in range(DEPTH):
    parts = _seg_partials(h, edge_index, spad, dpad)
    if i == 0:
      h, pp, mx, mo = _layer1_tc(h, parts, gin_W1[i], gin_b1[i],
                                 gin_bn_g[i], gin_bn_b[i], gin_W2[i],
                                 gin_b2[i], norm_g[i], norm_b[i], q0, qs[i])
    else:
      h, pp, mo = _layer23_tc(parts, gin_W1[i], gin_b1[i],
                              gin_bn_g[i], gin_bn_b[i], gin_W2[i],
                              gin_b2[i], norm_g[i], norm_b[i], qs[i])
    pps.append(pp)
    mos.append(mo)
  pi, v = _head_tc(pps[0], pps[1], pps[2], mx, mos[0], mos[1], mos[2],
                   v0, vs[0], vs[1], vs[2], c_pi, c_v)
  return (pi, v)


# speedup vs baseline: 1.3981x; 1.0332x over previous
import functools

import jax
import jax.numpy as jnp
from jax import lax
from jax.experimental import pallas as pl
from jax.experimental.pallas import tpu as pltpu
from jax.experimental.pallas import tpu_sc as plsc

N = 10000
E = 320000
D = 128
DEPTH = 3
BATCH = 10
NV = 3
GPN = N // BATCH

NC = 2
NS = 16
NW = NC * NS
CH = 128
CPW = 80
EPC = NW * CPW * CH
NP = 10240
RPT = NP // NS
ZR = 128
NRING = 4


def _seg_partials(h, ei, spad, dpad):
  mesh = plsc.VectorSubcoreMesh(core_axis_name="c", subcore_axis_name="s")

  @functools.partial(
      pl.kernel,
      mesh=mesh,
      out_type=jax.ShapeDtypeStruct((NC * NP, D), jnp.float32),
      scratch_types=[
          pltpu.VMEM((NRING, CH), jnp.int32),
          pltpu.VMEM((NRING, CH), jnp.int32),
          pltpu.VMEM((CH, D), jnp.float32),
          pltpu.VMEM((CH, D), jnp.float32),
          pltpu.VMEM_SHARED((NP, D), jnp.float32),
          pltpu.SemaphoreType.DMA,
          pltpu.SemaphoreType.DMA,
          pltpu.SemaphoreType.DMA,
      ],
  )
  def k(h_hbm, ei_hbm, spad_hbm, dpad_hbm, out_hbm,
        sidx, didx, rows0, rows1, acc, sem_i, sem0, sem1):
    cid = lax.axis_index("c")
    sid = lax.axis_index("s")
    wid = sid * NC + cid
    ebase = wid * CPW * CH

    def ifire(j):
      s = j % NRING
      off = ebase + j * CH

      @pl.when(off < E)
      def _():
        pltpu.make_async_copy(
            ei_hbm.at[0, pl.ds(off, CH)], sidx.at[s], sem_i).start()
        pltpu.make_async_copy(
            ei_hbm.at[1, pl.ds(off, CH)], didx.at[s], sem_i).start()

      @pl.when(off >= E)
      def _():
        pltpu.make_async_copy(
            spad_hbm.at[pl.ds(off - E, CH)], sidx.at[s], sem_i).start()
        pltpu.make_async_copy(
            dpad_hbm.at[pl.ds(off - E, CH)], didx.at[s], sem_i).start()

    def idrain(j):
      s = j % NRING
      pltpu.make_async_copy(
          spad_hbm.at[pl.ds(0, CH)], sidx.at[s], sem_i).wait()
      pltpu.make_async_copy(
          spad_hbm.at[pl.ds(0, CH)], didx.at[s], sem_i).wait()

    for j in range(2):
      ifire(j)

    @pl.when(cid == 0)
    def _():

      def hc(i, carry):
        r0 = sid * RPT + i * ZR
        pltpu.sync_copy(h_hbm.at[pl.ds(r0, ZR)], acc.at[pl.ds(r0, ZR)])
        return carry

      @pl.when(sid < NS - 1)
      def _():
        lax.fori_loop(0, RPT // ZR, hc, 0)

      @pl.when(sid == NS - 1)
      def _():
        lax.fori_loop(0, (N - (NS - 1) * RPT) // ZR, hc, 0)
        nrem = N - (NS - 1) * RPT - ((N - (NS - 1) * RPT) // ZR) * ZR
        if nrem:
          base = (NS - 1) * RPT + ((N - (NS - 1) * RPT) // ZR) * ZR
          pltpu.sync_copy(h_hbm.at[pl.ds(base, nrem)],
                          acc.at[pl.ds(base, nrem)])

    @pl.when(cid == 1)
    def _():
      zeros16 = jnp.zeros((16,), jnp.float32)

      def zb(i, carry):
        r = i // (D // 16)
        c = i % (D // 16)
        rows0[r, pl.ds(c * 16, 16)] = zeros16
        return carry

      lax.fori_loop(0, CH * (D // 16), zb, 0)

      def zc(i, carry):
        pltpu.sync_copy(rows0, acc.at[pl.ds(sid * RPT + i * ZR, ZR)])
        return carry

      lax.fori_loop(0, RPT // ZR, zc, 0)

    for j in range(2):
      idrain(j)
    plsc.subcore_barrier()

    def gstart(j, buf, sem):
      pltpu.make_async_copy(h_hbm.at[sidx.at[j % NRING]], buf, sem).start()

    def gwait(j, buf, sem):
      pltpu.make_async_copy(h_hbm.at[sidx.at[j % NRING]], buf, sem).wait()

    gstart(0, rows0, sem0)
    last = CPW // 2 - 1

    def body(jj, carry):
      j = jj * 2

      @pl.when(jj < last)
      def _():
        ifire(j + 2)
        ifire(j + 3)

      gstart(j + 1, rows1, sem1)
      gwait(j, rows0, sem0)
      pltpu.sync_copy(rows0, acc.at[didx.at[j % NRING]], add=True)

      @pl.when(jj < last)
      def _():
        idrain(j + 2)
        idrain(j + 3)
        gstart(j + 2, rows0, sem0)

      gwait(j + 1, rows1, sem1)
      pltpu.sync_copy(rows1, acc.at[didx.at[(j + 1) % NRING]], add=True)
      return carry

    lax.fori_loop(0, CPW // 2, body, 0)
    plsc.subcore_barrier()

    def wo(i, carry):
      r0 = sid * RPT + i * ZR
      pltpu.sync_copy(acc.at[pl.ds(r0, ZR)],
                      out_hbm.at[pl.ds(cid * NP + r0, ZR)])
      return carry

    lax.fori_loop(0, RPT // ZR, wo, 0)

  return k(h, ei, spad, dpad)


def _mlp(z, w1_ref, b1_ref, g1_ref, bb1_ref, w2_ref, b2_ref, g2_ref, bb2_ref):
  z = jnp.dot(z, w1_ref[...], preferred_element_type=jnp.float32) + b1_ref[...]
  mu = jnp.mean(z, axis=0, keepdims=True)
  var = jnp.mean(jnp.square(z - mu), axis=0, keepdims=True)
  z = (z - mu) * lax.rsqrt(var + 1e-5) * g1_ref[...] + bb1_ref[...]
  z = jnp.maximum(z, 0.0)
  z = jnp.dot(z, w2_ref[...], preferred_element_type=jnp.float32) + b2_ref[...]
  mu = jnp.mean(z, axis=0, keepdims=True)
  var = jnp.mean(jnp.square(z - mu), axis=0, keepdims=True)
  z = (z - mu) * lax.rsqrt(var + 1e-5) * g2_ref[...] + bb2_ref[...]
  return jnp.maximum(z, 0.0)


def _layer1_body(h_ref, parts_ref, w1_ref, b1_ref, g1_ref, bb1_ref,
                 w2_ref, b2_ref, g2_ref, bb2_ref, qa_ref, qb_ref,
                 o_ref, pp_ref, mi_ref, mo_ref):
  hin = h_ref[...]
  hout = _mlp(parts_ref[0] + parts_ref[1], w1_ref, b1_ref, g1_ref, bb1_ref,
              w2_ref, b2_ref, g2_ref, bb2_ref)
  o_ref[...] = hout
  hin_r = hin.reshape(BATCH, GPN, D)
  hout_r = hout.reshape(BATCH, GPN, D)
  pp_ref[...] = (
      jnp.matmul(hin_r, qa_ref[...], preferred_element_type=jnp.float32)
      + jnp.matmul(hout_r, qb_ref[...], preferred_element_type=jnp.float32)
  )[..., 0]
  mi_ref[...] = jnp.mean(hin_r, axis=1)
  mo_ref[...] = jnp.mean(hout_r, axis=1)


def _layer23_body(parts_ref, w1_ref, b1_ref, g1_ref, bb1_ref,
                  w2_ref, b2_ref, g2_ref, bb2_ref, qb_ref,
                  o_ref, pp_ref, mo_ref):
  hout = _mlp(parts_ref[0] + parts_ref[1], w1_ref, b1_ref, g1_ref, bb1_ref,
              w2_ref, b2_ref, g2_ref, bb2_ref)
  o_ref[...] = hout
  hout_r = hout.reshape(BATCH, GPN, D)
  pp_ref[...] = jnp.matmul(
      hout_r, qb_ref[...], preferred_element_type=jnp.float32)[..., 0]
  mo_ref[...] = jnp.mean(hout_r, axis=1)


def _wsp(s):
  return pl.BlockSpec(s, lambda i: (0,) * len(s))


_WSPECS = [_wsp((D, D)), _wsp((1, D)), _wsp((1, D)), _wsp((1, D)),
           _wsp((D, D)), _wsp((1, D)), _wsp((1, D)), _wsp((1, D))]


def _layer1_tc(h, parts, w1, b1, g1, bb1, w2, b2, g2, bb2, qa, qb):
  r = lambda a: a.reshape(1, D)
  return pl.pallas_call(
      _layer1_body,
      grid=(1,),
      in_specs=[_wsp((N, D)), _wsp((2, N, D))] + _WSPECS
      + [_wsp((D, 1)), _wsp((D, 1))],
      out_specs=[_wsp((N, D)), _wsp((BATCH, GPN)), _wsp((BATCH, D)),
                 _wsp((BATCH, D))],
      out_shape=[
          jax.ShapeDtypeStruct((N, D), jnp.float32),
          jax.ShapeDtypeStruct((BATCH, GPN), jnp.float32),
          jax.ShapeDtypeStruct((BATCH, D), jnp.float32),
          jax.ShapeDtypeStruct((BATCH, D), jnp.float32),
      ],
  )(h, parts.reshape(2, NP, D), w1, r(b1), r(g1), r(bb1),
    w2, r(b2), r(g2), r(bb2), qa, qb)


def _layer23_tc(parts, w1, b1, g1, bb1, w2, b2, g2, bb2, qb):
  r = lambda a: a.reshape(1, D)
  return pl.pallas_call(
      _layer23_body,
      grid=(1,),
      in_specs=[_wsp((2, N, D))] + _WSPECS + [_wsp((D, 1))],
      out_specs=[_wsp((N, D)), _wsp((BATCH, GPN)), _wsp((BATCH, D))],
      out_shape=[
          jax.ShapeDtypeStruct((N, D), jnp.float32),
          jax.ShapeDtypeStruct((BATCH, GPN), jnp.float32),
          jax.ShapeDtypeStruct((BATCH, D), jnp.float32),
      ],
  )(parts.reshape(2, NP, D), w1, r(b1), r(g1), r(bb1),
    w2, r(b2), r(g2), r(bb2), qb)


def _head_body(pp1_ref, pp2_ref, pp3_ref, mx_ref, m1_ref, m2_ref, m3_ref,
               v0_ref, v1_ref, v2_ref, v3_ref, cpi_ref, cv_ref,
               pi_ref, v_ref):
  s = pp1_ref[...] + pp2_ref[...] + pp3_ref[...] + cpi_ref[0, 0]
  m = jnp.max(s, axis=1, keepdims=True)
  lse = jnp.log(jnp.sum(jnp.exp(s - m), axis=1, keepdims=True)) + m
  pi_ref[...] = s - lse
  dot = lambda a, b: jnp.dot(a, b, preferred_element_type=jnp.float32)
  v = (dot(mx_ref[...], v0_ref[...]) + dot(m1_ref[...], v1_ref[...])
       + dot(m2_ref[...], v2_ref[...]) + dot(m3_ref[...], v3_ref[...])
       + cv_ref[...])
  vm = jnp.max(v, axis=1, keepdims=True)
  vlse = jnp.log(jnp.sum(jnp.exp(v - vm), axis=1, keepdims=True)) + vm
  v_ref[...] = v - vlse


def _head_tc(pp1, pp2, pp3, mx, m1, m2, m3, v0, v1, v2, v3, c_pi, c_v):
  wsp = lambda s: pl.BlockSpec(s, lambda i: (0,) * len(s))
  return pl.pallas_call(
      _head_body,
      grid=(1,),
      in_specs=[
          wsp((BATCH, GPN)), wsp((BATCH, GPN)), wsp((BATCH, GPN)),
          wsp((BATCH, D)), wsp((BATCH, D)), wsp((BATCH, D)), wsp((BATCH, D)),
          wsp((D, NV)), wsp((D, NV)), wsp((D, NV)), wsp((D, NV)),
          wsp((1, 1)), wsp((1, NV)),
      ],
      out_specs=[wsp((BATCH, GPN)), wsp((BATCH, NV))],
      out_shape=[
          jax.ShapeDtypeStruct((BATCH, GPN), jnp.float32),
          jax.ShapeDtypeStruct((BATCH, NV), jnp.float32),
      ],
  )(pp1, pp2, pp3,
    mx, m1, m2, m3, v0, v1, v2, v3, c_pi.reshape(1, 1), c_v.reshape(1, NV))


def kernel(x, edge_index, batch_size, gin_W1, gin_b1, gin_bn_g, gin_bn_b,
           gin_W2, gin_b2, norm_g, norm_b, jk_W, jk_b, lin_W, lin_b,
           v_W, v_b, pi_W, pi_b):
  pad = EPC - E
  ar = jnp.arange(pad, dtype=jnp.int32)
  spad = ar % N
  dpad = N + (ar % (NP - N))

  p_h = jk_W @ (lin_W[D:] @ pi_W)
  q0 = lin_W[:D] @ pi_W
  qs = [p_h[i * D:(i + 1) * D] for i in range(DEPTH)]
  c_pi = jk_b @ (lin_W[D:] @ pi_W) + lin_b @ pi_W + pi_b
  vh = jk_W @ (lin_W[D:] @ v_W)
  v0 = lin_W[:D] @ v_W
  vs = [vh[i * D:(i + 1) * D] for i in range(DEPTH)]
  c_v = jk_b @ (lin_W[D:] @ v_W) + lin_b @ v_W + v_b

  h = x
  pps, mos = [], []
  mx = None
  for i in range(DEPTH):
    parts = _seg_partials(h, edge_index, spad, dpad)
    if i == 0:
      h, pp, mx, mo = _layer1_tc(h, parts, gin_W1[i], gin_b1[i],
                                 gin_bn_g[i], gin_bn_b[i], gin_W2[i],
                                 gin_b2[i], norm_g[i], norm_b[i], q0, qs[i])
    else:
      h, pp, mo = _layer23_tc(parts, gin_W1[i], gin_b1[i],
                              gin_bn_g[i], gin_bn_b[i], gin_W2[i],
                              gin_b2[i], norm_g[i], norm_b[i], qs[i])
    pps.append(pp)
    mos.append(mo)
  pi, v = _head_tc(pps[0], pps[1], pps[2], mx, mos[0], mos[1], mos[2],
                   v0, vs[0], vs[1], vs[2], c_pi, c_v)
  return (pi, v)
